# Initial kernel scaffold; baseline (speedup 1.0000x reference)
import jax
import jax.numpy as jnp
from jax.experimental import pallas as pl


def kernel(alpha, laplacian, num_node, threshold, diff_vec, edge_index, W1, b1, W2, b2, Wf, bf):
    raise NotImplementedError("write your pallas kernel here")


# speedup vs baseline: 30.9641x; 30.9641x over previous
import functools

import jax
import jax.numpy as jnp
from jax import lax
from jax.experimental import pallas as pl
from jax.experimental.pallas import tpu as pltpu
from jax.experimental.pallas import tpu_sc as plsc

_N = 4096
_E = 65536
_D = 16
_K_ITERS = 16
_BM = 512
_NW = 32
_CHUNK = 128
_NCHUNK = _E // (_NW * _CHUNK)
_ROWS_PER_TILE = _N // 16


def _solver_body(alpha_ref, L_ref, V_ref, out_ref, y0, y1):
    k = pl.program_id(0)
    i = pl.program_id(1)
    alpha = alpha_ref[0, 0]

    @pl.when(jnp.logical_and(k == 0, i == 0))
    def _():
        y0[...] = V_ref[...]

    def step(src, dst):
        blk = V_ref[pl.ds(i * _BM, _BM), :] + alpha * jnp.dot(
            L_ref[...], src[...], preferred_element_type=jnp.float32)
        dst[pl.ds(i * _BM, _BM), :] = blk
        out_ref[...] = (1.0 - alpha) * blk

    @pl.when(k % 2 == 0)
    def _():
        step(y0, y1)

    @pl.when(k % 2 == 1)
    def _():
        step(y1, y0)


def _neumann_solve(alpha, laplacian, V):
    return pl.pallas_call(
        _solver_body,
        grid=(_K_ITERS, _N // _BM),
        in_specs=[
            pl.BlockSpec(memory_space=pltpu.SMEM),
            pl.BlockSpec((_BM, _N), lambda k, i: (i, 0)),
            pl.BlockSpec((_N, 8), lambda k, i: (0, 0)),
        ],
        out_specs=pl.BlockSpec((_BM, 8), lambda k, i: (i, 0)),
        out_shape=jax.ShapeDtypeStruct((_N, 8), jnp.float32),
        scratch_shapes=[
            pltpu.VMEM((_N, 8), jnp.float32),
            pltpu.VMEM((_N, 8), jnp.float32),
        ],
    )(jnp.reshape(alpha, (1, 1)), laplacian, V)


def _sc_scatter_body(src_hbm, dst_hbm, table_hbm, zeros_hbm, out_hbm,
                     srcv, dstv, rows, zrows, acc, sem):
    c = lax.axis_index("c")
    s = lax.axis_index("s")
    wid = s * 2 + c

    pltpu.sync_copy(zeros_hbm, zrows)
    pltpu.sync_copy(zrows, acc.at[pl.ds(s * _ROWS_PER_TILE, _ROWS_PER_TILE)])
    plsc.subcore_barrier()

    pltpu.sync_copy(src_hbm.at[wid], srcv)
    pltpu.sync_copy(dst_hbm.at[wid], dstv)

    def chunk(j, carry):
        pltpu.async_copy(table_hbm.at[srcv.at[j]], rows, sem).wait()
        pltpu.sync_copy(rows, acc.at[dstv.at[j]], add=True)
        return carry

    lax.fori_loop(0, _NCHUNK, chunk, 0)
    plsc.subcore_barrier()

    base = c * _N + s * _ROWS_PER_TILE
    pltpu.sync_copy(acc.at[pl.ds(s * _ROWS_PER_TILE, _ROWS_PER_TILE)],
                    out_hbm.at[pl.ds(base, _ROWS_PER_TILE)])


def _sc_scatter(src3, dst3, table, zeros_hbm):
    mesh = plsc.VectorSubcoreMesh(core_axis_name="c", subcore_axis_name="s")
    f = pl.kernel(
        _sc_scatter_body,
        out_type=jax.ShapeDtypeStruct((2 * _N, _D), jnp.float32),
        mesh=mesh,
        scratch_types=[
            pltpu.VMEM((_NCHUNK, _CHUNK), jnp.int32),
            pltpu.VMEM((_NCHUNK, _CHUNK), jnp.int32),
            pltpu.VMEM((_CHUNK, _D), jnp.float32),
            pltpu.VMEM((_ROWS_PER_TILE, _D), jnp.float32),
            pltpu.VMEM_SHARED((_N, _D), jnp.float32),
            pltpu.SemaphoreType.DMA,
        ],
        compiler_params=pltpu.CompilerParams(use_tc_tiling_on_sc=False),
    )
    return f(src3, dst3, table, zeros_hbm)


def _prep_body(degp_ref, Y_ref, dv_ref, table1_ref):
    deg = degp_ref[0:_N, 0:1] + degp_ref[_N:2 * _N, 0:1] + 1.0
    dinv = lax.rsqrt(deg)
    table1_ref[...] = jnp.concatenate(
        [dinv * dv_ref[...], dinv * Y_ref[:, 0:3], dinv,
         jnp.zeros((_N, 11), jnp.float32)], axis=1)


def _mid_body(praw_ref, table1_ref, W1_ref, b1_ref, W2_ref, Wf_ref,
              table2_ref, s_ref):
    dinv = table1_ref[:, 4:5]
    p_full = dinv * (praw_ref[0:_N, :] + praw_ref[_N:2 * _N, :]) \
        + dinv * table1_ref[...]
    p = p_full[:, 0:4]
    s = p_full[:, 4:5]
    z = jnp.dot(p, W1_ref[...].T, preferred_element_type=jnp.float32) \
        + s * b1_ref[...]
    h = jnp.maximum(z, 0.0)
    C = jnp.dot(Wf_ref[...], W2_ref[...], preferred_element_type=jnp.float32)
    g = jnp.dot(h, C.T, preferred_element_type=jnp.float32)
    table2_ref[...] = jnp.concatenate(
        [dinv * g, jnp.zeros((_N, 14), jnp.float32)], axis=1)
    s_ref[...] = s


def _final_body(qraw_ref, table1_ref, table2_ref, s_ref, Wf_ref, b2_ref,
                bf_ref, out_ref):
    dinv = table1_ref[:, 4:5]
    qsum = qraw_ref[0:_N, 0:2] + qraw_ref[_N:2 * _N, 0:2]
    sb = jnp.dot(b2_ref[...], Wf_ref[...].T,
                 preferred_element_type=jnp.float32)
    out_ref[...] = dinv * qsum + dinv * table2_ref[:, 0:2] \
        + s_ref[...] * sb + bf_ref[...]


def _tc_single(body, out_shapes, *args):
    return pl.pallas_call(
        body,
        out_shape=out_shapes,
    )(*args)


def kernel(alpha, laplacian, num_node, threshold, diff_vec, edge_index,
           W1, b1, W2, b2, Wf, bf):
    n = diff_vec.shape[0]
    v = diff_vec.astype(jnp.float32)
    V3 = jnp.where(v < threshold, threshold, v)
    V4 = jnp.where(v >= threshold, threshold, v)
    V = jnp.concatenate(
        [v[:, None], V3[:, None], V4[:, None], jnp.zeros((n, 5), jnp.float32)],
        axis=1)

    Y = _neumann_solve(alpha, laplacian, V)

    src3 = edge_index[0].reshape(_NW, _NCHUNK, _CHUNK)
    dst3 = edge_index[1].reshape(_NW, _NCHUNK, _CHUNK)
    zeros_hbm = jnp.zeros((_ROWS_PER_TILE, _D), jnp.float32)
    ones_table = jnp.ones((_N, _D), jnp.float32)

    degp = _sc_scatter(src3, src3, ones_table, zeros_hbm)

    table1 = _tc_single(
        _prep_body, jax.ShapeDtypeStruct((_N, _D), jnp.float32),
        degp, Y, v[:, None])

    praw = _sc_scatter(src3, dst3, table1, zeros_hbm)

    table2, s = _tc_single(
        _mid_body,
        [jax.ShapeDtypeStruct((_N, _D), jnp.float32),
         jax.ShapeDtypeStruct((_N, 1), jnp.float32)],
        praw, table1, W1, jnp.reshape(b1, (1, 128)), W2, Wf)

    qraw = _sc_scatter(src3, dst3, table2, zeros_hbm)

    out = _tc_single(
        _final_body, jax.ShapeDtypeStruct((_N, 2), jnp.float32),
        qraw, table1, table2, s, Wf, jnp.reshape(b2, (1, 128)),
        jnp.reshape(bf, (1, 2)))

    return out + (jnp.asarray(num_node) - n).astype(out.dtype)
